# baseline (speedup 1.0000x reference)
import functools

import jax
import jax.numpy as jnp
from jax import lax
from jax.experimental import pallas as pl
from jax.experimental.pallas import tpu as pltpu

_TOPK = 4096
_NMAX = 32768
_THR = 0.05
_EPS = 1e-12


def _norm_kernel(m_ref, mn_ref):
    mm = m_ref[...]
    ss = mm[0, :] * mm[0, :]
    for i in range(1, 64):
        ss = ss + mm[i, :] * mm[i, :]
    denom = jnp.maximum(jnp.sqrt(ss), _EPS)
    mn_ref[...] = mm / denom[None, :]


def _cubic_w(t):
    a = -0.75
    t1 = t + 1.0
    t2 = 1.0 - t
    t3 = 2.0 - t
    w0 = a * t1 ** 3 - 5 * a * t1 ** 2 + 8 * a * t1 - 4 * a
    w1 = (a + 2) * t ** 3 - (a + 3) * t ** 2 + 1.0
    w2 = (a + 2) * t2 ** 3 - (a + 3) * t2 ** 2 + 1.0
    w3 = a * t3 ** 3 - 5 * a * t3 ** 2 + 8 * a * t3 - 4 * a
    return [w0, w1, w2, w3]


def _feats_kernel(x_ref, y_ref, s_ref, m_ref, sel_ref, out_ref):
    posx = x_ref[...]
    posy = y_ref[...]

    def coords(pos):
        g = 2.0 * pos / jnp.float32(639.0) - 1.0
        f = ((g + 1.0) * 80.0 - 1.0) / 2.0
        f0 = jnp.floor(f)
        return f0.astype(jnp.int32), f - f0

    x0, txp = coords(posx)
    y0, typ = coords(posy)
    wx = _cubic_w(txp)
    wy = _cubic_w(typ)

    io80 = lax.broadcasted_iota(jnp.int32, (512, 80), 1)
    wy_m = jnp.zeros((512, 80), jnp.float32)
    for dy in range(4):
        wy_m = wy_m + jnp.where(io80 == (y0 + (dy - 1)), wy[dy],
                                jnp.float32(0.0))

    g_big = jnp.dot(wy_m, m_ref[...],
                    preferred_element_type=jnp.float32,
                    precision=lax.Precision.HIGHEST)

    iolane = lax.broadcasted_iota(jnp.int32, (512, 5120), 1) // 64
    wx_m = jnp.zeros((512, 5120), jnp.float32)
    for dx in range(4):
        wx_m = wx_m + jnp.where(iolane == (x0 + (dx - 1)), wx[dx],
                                jnp.float32(0.0))

    z = g_big * wx_m
    feats = jnp.dot(z, sel_ref[...],
                    preferred_element_type=jnp.float32,
                    precision=lax.Precision.HIGHEST)

    nrm = jnp.sqrt(jnp.sum(feats * feats, axis=1, keepdims=True))
    feats = feats / jnp.maximum(nrm, jnp.float32(_EPS))

    out_ref[:, 0:1] = posx
    out_ref[:, 1:2] = posy
    out_ref[:, 2:3] = s_ref[...]
    out_ref[:, 3:67] = feats


def _ngrid(pos, H, W):
    return 2.0 * pos / jnp.array([W - 1, H - 1], dtype=pos.dtype) - 1.0


def _unn(g, size):
    return ((g + 1.0) * size - 1.0) / 2.0


def _g2d(img, iy, ix):
    Hin, Win = img.shape[1], img.shape[2]
    inb = (iy >= 0) & (iy < Hin) & (ix >= 0) & (ix < Win)
    iyc = jnp.clip(iy, 0, Hin - 1)
    ixc = jnp.clip(ix, 0, Win - 1)
    v = img[:, iyc, ixc]
    return v * inb[None, :].astype(img.dtype)


def _crd(img, pos, H, W):
    Hin, Win = img.shape[2], img.shape[3]
    g = _ngrid(pos.astype(img.dtype), H, W)
    ix = _unn(g[..., 0], Win)
    iy = _unn(g[..., 1], Hin)
    return ix, iy


def _gs_nearest(img, pos, H, W):
    ix, iy = _crd(img, pos, H, W)
    ixn = jnp.round(ix).astype(jnp.int32)
    iyn = jnp.round(iy).astype(jnp.int32)
    out = jax.vmap(_g2d)(img, iyn, ixn)
    return jnp.transpose(out, (0, 2, 1))


def _gs_bilinear(img, pos, H, W):
    ix, iy = _crd(img, pos, H, W)
    x0 = jnp.floor(ix)
    y0 = jnp.floor(iy)
    tx = (ix - x0)[:, None, :]
    ty = (iy - y0)[:, None, :]
    x0i = x0.astype(jnp.int32)
    y0i = y0.astype(jnp.int32)

    def samp(dy, dx):
        return jax.vmap(_g2d)(img, y0i + dy, x0i + dx)

    out = (samp(0, 0) * (1 - tx) * (1 - ty) + samp(0, 1) * tx * (1 - ty)
           + samp(1, 0) * (1 - tx) * ty + samp(1, 1) * tx * ty)
    return jnp.transpose(out, (0, 2, 1))


@functools.partial(jax.jit, static_argnums=())
def kernel(M1, K1, H1):
    M1r = M1.reshape(64, 6400)
    m1n = pl.pallas_call(
        _norm_kernel,
        out_shape=jax.ShapeDtypeStruct((64, 6400), jnp.float32),
    )(M1r)

    sc = jax.nn.softmax(K1, axis=1)[:, :64]
    B = sc.shape[0]
    Hc, Wc = sc.shape[2], sc.shape[3]
    hm = jnp.transpose(sc, (0, 2, 3, 1)).reshape(B, Hc, Wc, 8, 8)
    hm = jnp.transpose(hm, (0, 1, 3, 2, 4)).reshape(B, 1, Hc * 8, Wc * 8)
    lm = jax.lax.reduce_window(hm, -jnp.inf, jax.lax.max, (1, 1, 5, 5),
                               (1, 1, 1, 1),
                               ((0, 0), (0, 0), (2, 2), (2, 2)))
    mask = (hm == lm) & (hm > _THR)
    ys, xs = jnp.nonzero(mask[0, 0], size=_NMAX, fill_value=0)
    mkpts = jnp.stack([xs, ys], axis=-1)[None].astype(jnp.int32)
    mkf = mkpts.astype(jnp.float32)
    s_near = _gs_nearest(hm, mkf, 640, 640)
    s_bil = _gs_bilinear(H1, mkf, 640, 640)
    scores = (s_near * s_bil)[..., 0]
    scores = jnp.where(jnp.all(mkpts == 0, axis=-1), -1.0, scores)
    idxs = jnp.argsort(-scores, axis=-1)
    mx = jnp.take_along_axis(mkpts[..., 0], idxs, axis=-1)[:, :_TOPK]
    my = jnp.take_along_axis(mkpts[..., 1], idxs, axis=-1)[:, :_TOPK]
    sc_top = jnp.take_along_axis(scores, idxs, axis=-1)[:, :_TOPK]

    xi = mx.astype(jnp.float32).reshape(_TOPK, 1)
    yi = my.astype(jnp.float32).reshape(_TOPK, 1)
    st = sc_top.reshape(_TOPK, 1)

    m1nt = (m1n.reshape(64, 80, 80)
               .transpose(1, 2, 0)
               .reshape(80, 5120))
    io = jnp.arange(5120, dtype=jnp.int32)
    sel = (io[:, None] % 64 == jnp.arange(64, dtype=jnp.int32)[None, :])
    sel = sel.astype(jnp.float32)

    out = pl.pallas_call(
        _feats_kernel,
        grid=(8,),
        in_specs=[
            pl.BlockSpec((512, 1), lambda i: (i, 0)),
            pl.BlockSpec((512, 1), lambda i: (i, 0)),
            pl.BlockSpec((512, 1), lambda i: (i, 0)),
            pl.BlockSpec((80, 5120), lambda i: (0, 0)),
            pl.BlockSpec((5120, 64), lambda i: (0, 0)),
        ],
        out_specs=pl.BlockSpec((512, 67), lambda i: (i, 0)),
        out_shape=jax.ShapeDtypeStruct((_TOPK, 67), jnp.float32),
    )(xi, yi, st, m1nt, sel)

    return out.reshape(1, _TOPK, 67)


# speedup vs baseline: 1.0217x; 1.0217x over previous
import functools

import jax
import jax.numpy as jnp
from jax import lax
from jax.experimental import pallas as pl
from jax.experimental.pallas import tpu as pltpu

_TOPK = 4096
_NMAX = 32768
_THR = 0.05
_EPS = 1e-12


def _norm_kernel(m_ref, mn_ref):
    mm = m_ref[...]
    ss = mm[0, :] * mm[0, :]
    for i in range(1, 64):
        ss = ss + mm[i, :] * mm[i, :]
    denom = jnp.maximum(jnp.sqrt(ss), _EPS)
    mn_ref[...] = mm / denom[None, :]


def _cubic_w(t):
    a = -0.75
    t1 = t + 1.0
    t2 = 1.0 - t
    t3 = 2.0 - t
    w0 = a * t1 ** 3 - 5 * a * t1 ** 2 + 8 * a * t1 - 4 * a
    w1 = (a + 2) * t ** 3 - (a + 3) * t ** 2 + 1.0
    w2 = (a + 2) * t2 ** 3 - (a + 3) * t2 ** 2 + 1.0
    w3 = a * t3 ** 3 - 5 * a * t3 ** 2 + 8 * a * t3 - 4 * a
    return [w0, w1, w2, w3]


def _feats_kernel(x_ref, y_ref, s_ref, m_ref, sel_ref, out_ref):
    posx = x_ref[...]
    posy = y_ref[...]

    def coords(pos):
        g = 2.0 * pos / jnp.float32(639.0) - 1.0
        f = ((g + 1.0) * 80.0 - 1.0) / 2.0
        f0 = jnp.floor(f)
        return f0.astype(jnp.int32), f - f0

    x0, txp = coords(posx)
    y0, typ = coords(posy)
    wx = _cubic_w(txp)
    wy = _cubic_w(typ)

    io80 = lax.broadcasted_iota(jnp.int32, (512, 80), 1)
    wy_m = jnp.zeros((512, 80), jnp.float32)
    for dy in range(4):
        wy_m = wy_m + jnp.where(io80 == (y0 + (dy - 1)), wy[dy],
                                jnp.float32(0.0))

    g_big = jnp.dot(wy_m, m_ref[...],
                    preferred_element_type=jnp.float32,
                    precision=lax.Precision.HIGHEST)

    iolane = lax.broadcasted_iota(jnp.int32, (512, 5120), 1) // 64
    wx_m = jnp.zeros((512, 5120), jnp.float32)
    for dx in range(4):
        wx_m = wx_m + jnp.where(iolane == (x0 + (dx - 1)), wx[dx],
                                jnp.float32(0.0))

    z = g_big * wx_m
    feats = jnp.dot(z, sel_ref[...],
                    preferred_element_type=jnp.float32,
                    precision=lax.Precision.HIGHEST)

    nrm = jnp.sqrt(jnp.sum(feats * feats, axis=1, keepdims=True))
    feats = feats / jnp.maximum(nrm, jnp.float32(_EPS))

    out_ref[:, 0:1] = posx
    out_ref[:, 1:2] = posy
    out_ref[:, 2:3] = s_ref[...]
    out_ref[:, 3:67] = feats


def _ngrid(pos, H, W):
    return 2.0 * pos / jnp.array([W - 1, H - 1], dtype=pos.dtype) - 1.0


def _unn(g, size):
    return ((g + 1.0) * size - 1.0) / 2.0


def _g2d(img, iy, ix):
    Hin, Win = img.shape[1], img.shape[2]
    inb = (iy >= 0) & (iy < Hin) & (ix >= 0) & (ix < Win)
    iyc = jnp.clip(iy, 0, Hin - 1)
    ixc = jnp.clip(ix, 0, Win - 1)
    v = img[:, iyc, ixc]
    return v * inb[None, :].astype(img.dtype)


def _crd(img, pos, H, W):
    Hin, Win = img.shape[2], img.shape[3]
    g = _ngrid(pos.astype(img.dtype), H, W)
    ix = _unn(g[..., 0], Win)
    iy = _unn(g[..., 1], Hin)
    return ix, iy


def _gs_nearest(img, pos, H, W):
    ix, iy = _crd(img, pos, H, W)
    ixn = jnp.round(ix).astype(jnp.int32)
    iyn = jnp.round(iy).astype(jnp.int32)
    out = jax.vmap(_g2d)(img, iyn, ixn)
    return jnp.transpose(out, (0, 2, 1))


def _gs_bilinear(img, pos, H, W):
    ix, iy = _crd(img, pos, H, W)
    x0 = jnp.floor(ix)
    y0 = jnp.floor(iy)
    tx = (ix - x0)[:, None, :]
    ty = (iy - y0)[:, None, :]
    x0i = x0.astype(jnp.int32)
    y0i = y0.astype(jnp.int32)

    def samp(dy, dx):
        return jax.vmap(_g2d)(img, y0i + dy, x0i + dx)

    out = (samp(0, 0) * (1 - tx) * (1 - ty) + samp(0, 1) * tx * (1 - ty)
           + samp(1, 0) * (1 - tx) * ty + samp(1, 1) * tx * ty)
    return jnp.transpose(out, (0, 2, 1))


@functools.partial(jax.jit, static_argnums=())
def kernel(M1, K1, H1):
    M1r = M1.reshape(64, 6400)
    m1n = pl.pallas_call(
        _norm_kernel,
        out_shape=jax.ShapeDtypeStruct((64, 6400), jnp.float32),
    )(M1r)

    sc = jax.nn.softmax(K1, axis=1)[:, :64]
    B = sc.shape[0]
    Hc, Wc = sc.shape[2], sc.shape[3]
    hm = jnp.transpose(sc, (0, 2, 3, 1)).reshape(B, Hc, Wc, 8, 8)
    hm = jnp.transpose(hm, (0, 1, 3, 2, 4)).reshape(B, 1, Hc * 8, Wc * 8)
    lm = jax.lax.reduce_window(hm, -jnp.inf, jax.lax.max, (1, 1, 5, 5),
                               (1, 1, 1, 1),
                               ((0, 0), (0, 0), (2, 2), (2, 2)))
    mask = (hm == lm) & (hm > _THR)
    ys, xs = jnp.nonzero(mask[0, 0], size=_NMAX, fill_value=0)
    mkpts = jnp.stack([xs, ys], axis=-1)[None].astype(jnp.int32)
    mkf = mkpts.astype(jnp.float32)
    s_near = _gs_nearest(hm, mkf, 640, 640)
    s_bil = _gs_bilinear(H1, mkf, 640, 640)
    scores = (s_near * s_bil)[..., 0]
    scores = jnp.where(jnp.all(mkpts == 0, axis=-1), -1.0, scores)
    sc_top, idxs = lax.top_k(scores, _TOPK)
    mx = jnp.take_along_axis(mkpts[..., 0], idxs, axis=-1)
    my = jnp.take_along_axis(mkpts[..., 1], idxs, axis=-1)

    xi = mx.astype(jnp.float32).reshape(_TOPK, 1)
    yi = my.astype(jnp.float32).reshape(_TOPK, 1)
    st = sc_top.reshape(_TOPK, 1)

    m1nt = (m1n.reshape(64, 80, 80)
               .transpose(1, 2, 0)
               .reshape(80, 5120))
    io = jnp.arange(5120, dtype=jnp.int32)
    sel = (io[:, None] % 64 == jnp.arange(64, dtype=jnp.int32)[None, :])
    sel = sel.astype(jnp.float32)

    out = pl.pallas_call(
        _feats_kernel,
        grid=(8,),
        in_specs=[
            pl.BlockSpec((512, 1), lambda i: (i, 0)),
            pl.BlockSpec((512, 1), lambda i: (i, 0)),
            pl.BlockSpec((512, 1), lambda i: (i, 0)),
            pl.BlockSpec((80, 5120), lambda i: (0, 0)),
            pl.BlockSpec((5120, 64), lambda i: (0, 0)),
        ],
        out_specs=pl.BlockSpec((512, 67), lambda i: (i, 0)),
        out_shape=jax.ShapeDtypeStruct((_TOPK, 67), jnp.float32),
    )(xi, yi, st, m1nt, sel)

    return out.reshape(1, _TOPK, 67)
